# Initial kernel scaffold; baseline (speedup 1.0000x reference)
import jax
import jax.numpy as jnp
from jax.experimental import pallas as pl


def kernel(conditioning_states, forcings, scaled_noisy_target, sigmas, grid_positions, mesh_positions, g2m_senders, g2m_receivers, g2m_features, m2g_senders, m2g_receivers, m2g_features, mesh_neighbors, params):
    raise NotImplementedError("write your pallas kernel here")


# speedup vs baseline: 1.2176x; 1.2176x over previous
import functools

import numpy as np
import jax
import jax.numpy as jnp
from jax import lax
from jax.experimental import pallas as pl
from jax.experimental.pallas import tpu as pltpu

H = 90
W = 180
NG = H * W
NM = 10242
K_NBR = 16
D = 256
COND = 32
N_FREQ = 32
BASE_PERIOD = 16.0
N_HEADS = 8
DH = D // N_HEADS
FFW = 512


def _silu(x):
    return x / (1.0 + jnp.exp(-x))


def _ceil_to(n, m):
    return ((n + m - 1) // m) * m


def _pad_rows(x, m, value=0.0):
    n = x.shape[0]
    npad = _ceil_to(n, m)
    if npad == n:
        return x
    pad = [(0, npad - n)] + [(0, 0)] * (x.ndim - 1)
    return jnp.pad(x, pad, constant_values=value)


def _full_spec(shape):
    nd = len(shape)
    return pl.BlockSpec(shape, lambda i: (0,) * nd)


def _mlp_chain(x, layers, blk=512):
    n, din = x.shape
    xp = _pad_rows(x, blk)
    npad = xp.shape[0]
    grid = npad // blk
    acts = tuple(a for _, _, a in layers)
    dout = layers[-1][0].shape[1]

    wbs = []
    in_specs = [pl.BlockSpec((blk, din), lambda i: (i, 0))]
    for w, b, _ in layers:
        b2 = jnp.zeros((1, w.shape[1]), jnp.float32) if b is None else b.reshape(1, -1)
        wbs.extend([w, b2])
        in_specs.append(_full_spec(w.shape))
        in_specs.append(_full_spec(b2.shape))

    def body(x_ref, *refs):
        out_ref = refs[-1]
        h = x_ref[...]
        for i, act in enumerate(acts):
            h = jnp.dot(h, refs[2 * i][...], preferred_element_type=jnp.float32)
            h = h + refs[2 * i + 1][...]
            if act:
                h = _silu(h)
        out_ref[...] = h

    out = pl.pallas_call(
        body,
        grid=(grid,),
        in_specs=in_specs,
        out_specs=pl.BlockSpec((blk, dout), lambda i: (i, 0)),
        out_shape=jax.ShapeDtypeStruct((npad, dout), jnp.float32),
    )(xp, *wbs)
    return out[:n]


def _edge_mlp(sA, rB, ef, w0e, w1, b1, w2, b2, blk=512):
    n = sA.shape[0]
    grid = n // blk

    def body(sa_ref, rb_ref, ef_ref, w0_ref, w1_ref, b1_ref, w2_ref, b2_ref, out_ref):
        h0 = sa_ref[...] + rb_ref[...] + jnp.dot(
            ef_ref[...], w0_ref[...], preferred_element_type=jnp.float32)
        h0 = _silu(h0)
        h1 = _silu(jnp.dot(h0, w1_ref[...], preferred_element_type=jnp.float32) + b1_ref[...])
        out_ref[...] = jnp.dot(h1, w2_ref[...], preferred_element_type=jnp.float32) + b2_ref[...]

    de = ef.shape[1]
    return pl.pallas_call(
        body,
        grid=(grid,),
        in_specs=[
            pl.BlockSpec((blk, D), lambda i: (i, 0)),
            pl.BlockSpec((blk, D), lambda i: (i, 0)),
            pl.BlockSpec((blk, de), lambda i: (i, 0)),
            _full_spec(w0e.shape),
            _full_spec(w1.shape),
            _full_spec((1, D)),
            _full_spec(w2.shape),
            _full_spec((1, D)),
        ],
        out_specs=pl.BlockSpec((blk, D), lambda i: (i, 0)),
        out_shape=jax.ShapeDtypeStruct((n, D), jnp.float32),
    )(sA, rB, ef, w0e, w1, b1.reshape(1, -1), w2, b2.reshape(1, -1))


def _node_update(lat, agg, w0a, w0b, b0, w1, b1, w2, b2, fw, fb, cond8, blk=512):
    n = lat.shape[0]
    latp = _pad_rows(lat, blk)
    aggp = _pad_rows(agg, blk)
    npad = latp.shape[0]
    grid = npad // blk

    def body(lat_ref, agg_ref, cond_ref, w0a_ref, w0b_ref, b0_ref, w1_ref, b1_ref,
             w2_ref, b2_ref, fw_ref, fb_ref, out_ref):
        film = jnp.dot(cond_ref[...], fw_ref[...], preferred_element_type=jnp.float32) + fb_ref[...]
        scale = film[0:1, :D]
        shift = film[0:1, D:]
        latv = lat_ref[...]
        h = jnp.dot(latv, w0a_ref[...], preferred_element_type=jnp.float32)
        h = h + jnp.dot(agg_ref[...], w0b_ref[...], preferred_element_type=jnp.float32)
        h = _silu(h + b0_ref[...])
        h = _silu(jnp.dot(h, w1_ref[...], preferred_element_type=jnp.float32) + b1_ref[...])
        upd = jnp.dot(h, w2_ref[...], preferred_element_type=jnp.float32) + b2_ref[...]
        out_ref[...] = latv + upd * (1.0 + scale) + shift

    out = pl.pallas_call(
        body,
        grid=(grid,),
        in_specs=[
            pl.BlockSpec((blk, D), lambda i: (i, 0)),
            pl.BlockSpec((blk, D), lambda i: (i, 0)),
            _full_spec((8, COND)),
            _full_spec((D, D)),
            _full_spec((D, D)),
            _full_spec((1, D)),
            _full_spec((D, D)),
            _full_spec((1, D)),
            _full_spec((D, D)),
            _full_spec((1, D)),
            _full_spec((COND, 2 * D)),
            _full_spec((1, 2 * D)),
        ],
        out_specs=pl.BlockSpec((blk, D), lambda i: (i, 0)),
        out_shape=jax.ShapeDtypeStruct((npad, D), jnp.float32),
    )(latp, aggp, cond8, w0a, w0b, b0.reshape(1, -1), w1, b1.reshape(1, -1),
      w2, b2.reshape(1, -1), fw, fb.reshape(1, -1))
    return out[:n]


def _qkv(x, cond8, f1w, f1b, wq, bq, wk, bk, wv, bv, blk=512):
    n = x.shape[0]
    grid = n // blk

    def body(x_ref, cond_ref, f1w_ref, f1b_ref, wq_ref, bq_ref, wk_ref, bk_ref,
             wv_ref, bv_ref, q_ref, k_ref, v_ref):
        film = jnp.dot(cond_ref[...], f1w_ref[...], preferred_element_type=jnp.float32) + f1b_ref[...]
        s1 = film[0:1, :D]
        t1 = film[0:1, D:]
        xv = x_ref[...]
        mu = jnp.mean(xv, axis=-1, keepdims=True)
        var = jnp.mean((xv - mu) ** 2, axis=-1, keepdims=True)
        hv = (xv - mu) / jnp.sqrt(var + 1e-5) * (1.0 + s1) + t1
        q_ref[...] = jnp.dot(hv, wq_ref[...], preferred_element_type=jnp.float32) + bq_ref[...]
        k_ref[...] = jnp.dot(hv, wk_ref[...], preferred_element_type=jnp.float32) + bk_ref[...]
        v_ref[...] = jnp.dot(hv, wv_ref[...], preferred_element_type=jnp.float32) + bv_ref[...]

    row = pl.BlockSpec((blk, D), lambda i: (i, 0))
    outs = pl.pallas_call(
        body,
        grid=(grid,),
        in_specs=[
            row,
            _full_spec((8, COND)),
            _full_spec((COND, 2 * D)),
            _full_spec((1, 2 * D)),
            _full_spec((D, D)), _full_spec((1, D)),
            _full_spec((D, D)), _full_spec((1, D)),
            _full_spec((D, D)), _full_spec((1, D)),
        ],
        out_specs=[row, row, row],
        out_shape=[jax.ShapeDtypeStruct((n, D), jnp.float32)] * 3,
    )(x, cond8, f1w, f1b.reshape(1, -1), wq, bq.reshape(1, -1), wk, bk.reshape(1, -1),
      wv, bv.reshape(1, -1))
    return outs


def _attn_ffw(x, q, kn3, vn3, cond8, f2w, f2b, wo, bo, ff1w, ff1b, ff2w, ff2b, blk=256):
    n = x.shape[0]
    grid = n // blk

    def body(x_ref, q_ref, kn_ref, vn_ref, cond_ref, f2w_ref, f2b_ref, wo_ref, bo_ref,
             w1_ref, b1_ref, w2_ref, b2_ref, out_ref):
        r = lax.broadcasted_iota(jnp.int32, (D, N_HEADS), 0)
        c = lax.broadcasted_iota(jnp.int32, (D, N_HEADS), 1)
        hs = (r // DH == c).astype(jnp.float32)
        rt = lax.broadcasted_iota(jnp.int32, (N_HEADS, D), 0)
        ct = lax.broadcasted_iota(jnp.int32, (N_HEADS, D), 1)
        hst = (ct // DH == rt).astype(jnp.float32)

        qb = q_ref[...]
        scale = np.float32(1.0 / np.sqrt(DH))
        logits = []
        for kk in range(K_NBR):
            knk = kn_ref[:, kk, :]
            logits.append(jnp.dot(qb * knk, hs, preferred_element_type=jnp.float32) * scale)
        m = logits[0]
        for kk in range(1, K_NBR):
            m = jnp.maximum(m, logits[kk])
        exps = [jnp.exp(l - m) for l in logits]
        denom = exps[0]
        for kk in range(1, K_NBR):
            denom = denom + exps[kk]
        o = jnp.zeros((blk, D), jnp.float32)
        for kk in range(K_NBR):
            a = exps[kk] / denom
            o = o + jnp.dot(a, hst, preferred_element_type=jnp.float32) * vn_ref[:, kk, :]

        x1 = x_ref[...] + jnp.dot(o, wo_ref[...], preferred_element_type=jnp.float32) + bo_ref[...]

        film = jnp.dot(cond_ref[...], f2w_ref[...], preferred_element_type=jnp.float32) + f2b_ref[...]
        s2 = film[0:1, :D]
        t2 = film[0:1, D:]
        mu = jnp.mean(x1, axis=-1, keepdims=True)
        var = jnp.mean((x1 - mu) ** 2, axis=-1, keepdims=True)
        h2 = (x1 - mu) / jnp.sqrt(var + 1e-5) * (1.0 + s2) + t2
        ff = _silu(jnp.dot(h2, w1_ref[...], preferred_element_type=jnp.float32) + b1_ref[...])
        ff = jnp.dot(ff, w2_ref[...], preferred_element_type=jnp.float32) + b2_ref[...]
        out_ref[...] = x1 + ff

    row = pl.BlockSpec((blk, D), lambda i: (i, 0))
    out = pl.pallas_call(
        body,
        grid=(grid,),
        in_specs=[
            row,
            row,
            pl.BlockSpec((blk, K_NBR, D), lambda i: (i, 0, 0)),
            pl.BlockSpec((blk, K_NBR, D), lambda i: (i, 0, 0)),
            _full_spec((8, COND)),
            _full_spec((COND, 2 * D)),
            _full_spec((1, 2 * D)),
            _full_spec((D, D)), _full_spec((1, D)),
            _full_spec((D, FFW)), _full_spec((1, FFW)),
            _full_spec((FFW, D)), _full_spec((1, D)),
        ],
        out_specs=row,
        out_shape=jax.ShapeDtypeStruct((n, D), jnp.float32),
    )(x, q, kn3, vn3, cond8, f2w, f2b.reshape(1, -1), wo, bo.reshape(1, -1),
      ff1w, ff1b.reshape(1, -1), ff2w, ff2b.reshape(1, -1))
    return out


def _bipartite(p, sender_lat, receiver_lat, edge_feat, senders, receivers, cond8, n_recv):
    w0 = p['edge_mlp']['l0']['w']
    b0 = p['edge_mlp']['l0']['b']
    w0s, w0r, w0e = w0[:D], w0[D:2 * D], w0[2 * D:]

    A = _mlp_chain(sender_lat, [(w0s, None, False)])
    Br = _mlp_chain(receiver_lat, [(w0r, b0, False)])

    e = senders.shape[0]
    ep = _ceil_to(e, 512)
    senders_p = jnp.pad(senders, (0, ep - e))
    receivers_p = jnp.pad(receivers, (0, ep - e), constant_values=n_recv)
    ef_p = _pad_rows(edge_feat, 512)

    sA = jnp.take(A, senders_p, axis=0)
    rB = jnp.take(Br, jnp.where(receivers_p >= n_recv, 0, receivers_p), axis=0)

    em = p['edge_mlp']
    msg = _edge_mlp(sA, rB, ef_p, w0e,
                    em['l1']['w'], em['l1']['b'], em['l2']['w'], em['l2']['b'])

    agg = jax.ops.segment_sum(msg, receivers_p, num_segments=n_recv)

    nm = p['node_mlp']
    w0n = nm['l0']['w']
    return _node_update(receiver_lat, agg, w0n[:D], w0n[D:], nm['l0']['b'],
                        nm['l1']['w'], nm['l1']['b'], nm['l2']['w'], nm['l2']['b'],
                        p['film']['w'], p['film']['b'], cond8)


def _proc_layer(p, x, cond8, nbr_flat_p):
    q, k, v = _qkv(x, cond8, p['film1']['w'], p['film1']['b'],
                   p['wq']['w'], p['wq']['b'], p['wk']['w'], p['wk']['b'],
                   p['wv']['w'], p['wv']['b'])
    npad = x.shape[0]
    kn3 = jnp.take(k, nbr_flat_p, axis=0).reshape(npad, K_NBR, D)
    vn3 = jnp.take(v, nbr_flat_p, axis=0).reshape(npad, K_NBR, D)
    return _attn_ffw(x, q, kn3, vn3, cond8, p['film2']['w'], p['film2']['b'],
                     p['wo']['w'], p['wo']['b'], p['ff1']['w'], p['ff1']['b'],
                     p['ff2']['w'], p['ff2']['b'])


def kernel(conditioning_states, forcings, scaled_noisy_target, sigmas, grid_positions,
           mesh_positions, g2m_senders, g2m_receivers, g2m_features, m2g_senders,
           m2g_receivers, m2g_features, mesh_neighbors, params):
    b, s, c, h, w = conditioning_states.shape
    cond_flat = conditioning_states.reshape(s * c, h * w)
    grid_in = jnp.concatenate([cond_flat,
                               scaled_noisy_target.reshape(c, h * w),
                               forcings.reshape(-1, h * w)], 0)
    grid_nodes = jnp.concatenate([grid_in.T, grid_positions], -1)

    freqs = (2.0 * jnp.pi / BASE_PERIOD) * (2.0 ** jnp.arange(N_FREQ, dtype=jnp.float32))
    xf = jnp.log(sigmas)[:, None] * freqs[None, :]
    feats = jnp.concatenate([jnp.sin(xf), jnp.cos(xf)], -1)
    pn = params['noise']
    noise_emb = _mlp_chain(_pad_rows(feats, 8),
                           [(pn['l0']['w'], pn['l0']['b'], True),
                            (pn['l1']['w'], pn['l1']['b'], False)], blk=8)[:1]
    cond8 = _pad_rows(noise_emb, 8)

    ge = params['grid_embed']
    grid_lat = _mlp_chain(grid_nodes,
                          [(ge['l0']['w'], ge['l0']['b'], True),
                           (ge['l1']['w'], ge['l1']['b'], True),
                           (ge['l2']['w'], ge['l2']['b'], False)])
    me = params['mesh_embed']
    mesh_lat = _mlp_chain(mesh_positions,
                          [(me['l0']['w'], me['l0']['b'], True),
                           (me['l1']['w'], me['l1']['b'], True),
                           (me['l2']['w'], me['l2']['b'], False)])

    mesh_lat = _bipartite(params['g2m'], grid_lat, mesh_lat, g2m_features,
                          g2m_senders, g2m_receivers, cond8, NM)

    nmp = _ceil_to(NM, 512)
    xm = _pad_rows(mesh_lat, 512)
    nbr_flat = mesh_neighbors.reshape(-1).astype(jnp.int32)
    nbr_flat_p = jnp.pad(nbr_flat, (0, nmp * K_NBR - nbr_flat.shape[0]))
    for lp in params['proc']:
        xm = _proc_layer(lp, xm, cond8, nbr_flat_p)
    mesh_lat = xm[:NM]

    grid_lat = _bipartite(params['m2g'], mesh_lat, grid_lat, m2g_features,
                          m2g_senders, m2g_receivers, cond8, NG)

    oh = params['out_head']
    out = _mlp_chain(grid_lat,
                     [(oh['l0']['w'], oh['l0']['b'], True),
                      (oh['l1']['w'], oh['l1']['b'], True),
                      (oh['l2']['w'], oh['l2']['b'], False)])

    return out.T.reshape(1, c, h, w)
